# baseline (speedup 1.0000x reference)
import functools

import jax
import jax.numpy as jnp
from jax import lax
from jax.experimental import pallas as pl
from jax.experimental.pallas import tpu as pltpu
from jax.experimental.pallas import tpu_sc as plsc

BATCH = 4096
HIST = 200
EMBED_DIM = 64
SIZE = 128
NUM_EMB = 1000000

N = BATCH * HIST
GROW = 128
NW = 32
IDX_ROWS = N // GROW
ROWS_PER_W = IDX_ROWS // NW
IPW = ROWS_PER_W * GROW
LROWS = 208

BT = 16384
VLO = 31 * BT
NV0 = VLO
NV1 = NUM_EMB - VLO


def _mm_range(tableT, WT, blk0, nv):
    def mm(t_ref, w_ref, p_ref):
        p_ref[...] = lax.dot_general(
            t_ref[...], w_ref[...],
            (((0,), (0,)), ((), ())),
            preferred_element_type=jnp.float32,
        )

    return pl.pallas_call(
        mm,
        grid=((nv + BT - 1) // BT,),
        in_specs=[
            pl.BlockSpec((EMBED_DIM, BT), lambda i, b=blk0: (0, b + i)),
            pl.BlockSpec((EMBED_DIM, SIZE), lambda i: (0, 0)),
        ],
        out_specs=pl.BlockSpec((BT, SIZE), lambda i: (i, 0)),
        out_shape=jax.ShapeDtypeStruct((nv, SIZE), jnp.float32),
    )(tableT, WT)


_MESH = plsc.VectorSubcoreMesh(
    core_axis_name="c", subcore_axis_name="s", num_cores=2, num_subcores=16
)


def _sc_partition(idx2d):
    @functools.partial(
        pl.kernel,
        out_type=(
            jax.ShapeDtypeStruct((2, NW, LROWS, GROW), jnp.int32),
            jax.ShapeDtypeStruct((2, NW, LROWS, GROW), jnp.int32),
            jax.ShapeDtypeStruct((2, NW, 16), jnp.int32),
        ),
        mesh=_MESH,
        scratch_types=[
            pltpu.VMEM((ROWS_PER_W, GROW), jnp.int32),
            pltpu.VMEM((LROWS, GROW), jnp.int32),
            pltpu.VMEM((LROWS, GROW), jnp.int32),
            pltpu.VMEM((16,), jnp.int32),
        ],
        compiler_params=pltpu.CompilerParams(
            use_tc_tiling_on_sc=True, needs_layout_passes=False),
    )
    def k(idx_hbm, bidx_hbm, bpos_hbm, cnt_hbm, idx_v, ci, cp, cnt_v):
        wid = lax.axis_index("s") * 2 + lax.axis_index("c")
        base = wid * ROWS_PER_W
        pltpu.sync_copy(idx_hbm.at[pl.ds(base, ROWS_PER_W)], idx_v)
        lanes = lax.broadcasted_iota(jnp.int32, (16,), 0)
        posbase = wid * IPW

        for b in range(2):
            def scan_body(l, ptr, b=b):
                r = l >> 3
                c = (l & 7) << 4
                v = idx_v[r, pl.ds(c, 16)]
                if b == 0:
                    m = v < VLO
                    lv = v
                else:
                    m = v >= VLO
                    lv = v - VLO
                pos = posbase + (l << 4) + lanes
                offs = plsc.cumsum(jnp.where(m, 1, 0).astype(jnp.int32))
                dest = ptr + offs - 1
                row = dest >> 7
                col = dest & 127
                plsc.store_scatter(ci, [row, col], lv, mask=m)
                plsc.store_scatter(cp, [row, col], pos, mask=m)
                return ptr + jnp.max(offs)

            kcount = lax.fori_loop(0, IPW // 16, scan_body, jnp.int32(0))

            @pl.when(kcount > 0)
            def _():
                lrow = jnp.broadcast_to((kcount - 1) >> 7, (16,))
                lcol = jnp.broadcast_to((kcount - 1) & 127, (16,))
                padv = plsc.load_gather(ci, [lrow, lcol])
                padp = plsc.load_gather(cp, [lrow, lcol])
                for t in range(16):
                    dest = kcount + t * 16 + lanes
                    row = dest >> 7
                    col = dest & 127
                    plsc.store_scatter(ci, [row, col], padv)
                    plsc.store_scatter(cp, [row, col], padp)

            nch2 = ((kcount + 255) >> 8) * 2
            cnt_v[...] = jnp.broadcast_to(nch2, (16,))
            pltpu.sync_copy(ci, bidx_hbm.at[b, wid])
            pltpu.sync_copy(cp, bpos_hbm.at[b, wid])
            pltpu.sync_copy(cnt_v, cnt_hbm.at[b, wid])

    return k(idx2d)


def _sc_pass(P, bidx, bpos, counts, out_ref, b):
    @functools.partial(
        pl.kernel,
        out_type=(),
        mesh=_MESH,
        scratch_types=[
            pltpu.VMEM((LROWS, GROW), jnp.int32),
            pltpu.VMEM((LROWS, GROW), jnp.int32),
            pltpu.VMEM((16,), jnp.int32),
            pltpu.VMEM((GROW, SIZE), jnp.float32),
            pltpu.VMEM((GROW, SIZE), jnp.float32),
            pltpu.SemaphoreType.DMA,
            pltpu.SemaphoreType.DMA,
        ],
        compiler_params=pltpu.CompilerParams(
            use_tc_tiling_on_sc=True, needs_layout_passes=False),
    )
    def k(p_hbm, bidx_hbm, bpos_hbm, cnt_hbm, out_hbm, ci, cp, cnt_v,
          rows0, rows1, sem0, sem1):
        wid = lax.axis_index("s") * 2 + lax.axis_index("c")
        pltpu.sync_copy(cnt_hbm.at[b, wid], cnt_v)
        pltpu.sync_copy(bidx_hbm.at[b, wid], ci)
        pltpu.sync_copy(bpos_hbm.at[b, wid], cp)
        nt = jnp.max(cnt_v[...]) >> 1

        @pl.when(nt > 0)
        def _():
            pltpu.make_async_copy(p_hbm.at[ci.at[0]], rows0, sem0).start()

        def body(t2, carry):
            j0 = t2 * 2
            j1 = j0 + 1
            pltpu.make_async_copy(p_hbm.at[ci.at[j1]], rows1, sem1).start()
            pltpu.make_async_copy(p_hbm.at[ci.at[j0]], rows0, sem0).wait()
            pltpu.sync_copy(rows0, out_hbm.at[cp.at[j0]])

            @pl.when(t2 + 1 < nt)
            def _():
                pltpu.make_async_copy(
                    p_hbm.at[ci.at[j0 + 2]], rows0, sem0).start()

            pltpu.make_async_copy(p_hbm.at[ci.at[j1]], rows1, sem1).wait()
            pltpu.sync_copy(rows1, out_hbm.at[cp.at[j1]])
            return carry

        lax.fori_loop(0, nt, body, 0)

    k(P, bidx, bpos, counts, out_ref)


def kernel(input, table, W):
    idx2d = input.reshape(IDX_ROWS, GROW).astype(jnp.int32)
    tT = table.T
    wT = W.T
    bidx, bpos, counts = _sc_partition(idx2d)
    P0 = _mm_range(tT, wT, 0, NV0)
    P1 = _mm_range(tT, wT, 31, NV1)
    out_ref = jax.empty_ref(jax.ShapeDtypeStruct((N, SIZE), jnp.float32))
    _sc_pass(P0, bidx, bpos, counts, out_ref, 0)
    _sc_pass(P1, bidx, bpos, counts, out_ref, 1)
    out = jax.freeze(out_ref)
    return out.reshape(BATCH, HIST, SIZE)


# speedup vs baseline: 1.1682x; 1.1682x over previous
import functools

import jax
import jax.numpy as jnp
from jax import lax
from jax.experimental import pallas as pl
from jax.experimental.pallas import tpu as pltpu
from jax.experimental.pallas import tpu_sc as plsc

BATCH = 4096
HIST = 200
EMBED_DIM = 64
SIZE = 128
NUM_EMB = 1000000

N = BATCH * HIST
GROW = 128
NW = 32
IDX_ROWS = N // GROW
ROWS_PER_W = IDX_ROWS // NW


def _tc_project_table(tableT, WT):
    BT = 32768

    def mm(t_ref, w_ref, p_ref):
        p_ref[...] = lax.dot_general(
            t_ref[...], w_ref[...],
            (((0,), (0,)), ((), ())),
            preferred_element_type=jnp.float32,
        )

    return pl.pallas_call(
        mm,
        grid=((NUM_EMB + BT - 1) // BT,),
        in_specs=[
            pl.BlockSpec((EMBED_DIM, BT), lambda i: (0, i)),
            pl.BlockSpec((EMBED_DIM, SIZE), lambda i: (0, 0)),
        ],
        out_specs=pl.BlockSpec((BT, SIZE), lambda i: (i, 0)),
        out_shape=jax.ShapeDtypeStruct((NUM_EMB, SIZE), jnp.float32),
    )(tableT, WT)


def _sc_gather(P, idx2d):
    mesh = plsc.VectorSubcoreMesh(
        core_axis_name="c", subcore_axis_name="s", num_cores=2, num_subcores=16
    )

    @functools.partial(
        pl.kernel,
        out_type=jax.ShapeDtypeStruct((N, SIZE), jnp.float32),
        mesh=mesh,
        scratch_types=[
            pltpu.VMEM((ROWS_PER_W, GROW), jnp.int32),
            pltpu.VMEM((GROW, SIZE), jnp.float32),
            pltpu.VMEM((GROW, SIZE), jnp.float32),
            pltpu.SemaphoreType.DMA,
            pltpu.SemaphoreType.DMA,
        ],
        compiler_params=pltpu.CompilerParams(use_tc_tiling_on_sc=True),
    )
    def k(p_hbm, idx_hbm, out_hbm, idx_v, rows0, rows1, sem0, sem1):
        wid = lax.axis_index("s") * 2 + lax.axis_index("c")
        base = wid * ROWS_PER_W
        pltpu.sync_copy(idx_hbm.at[pl.ds(base, ROWS_PER_W)], idx_v)

        pltpu.make_async_copy(p_hbm.at[idx_v.at[0]], rows0, sem0).start()

        def body(t, carry):
            j0 = t * 2
            j1 = j0 + 1
            pltpu.make_async_copy(p_hbm.at[idx_v.at[j1]], rows1, sem1).start()
            pltpu.make_async_copy(p_hbm.at[idx_v.at[j0]], rows0, sem0).wait()
            pltpu.sync_copy(rows0, out_hbm.at[pl.ds((base + j0) * GROW, GROW)])

            @pl.when(t + 1 < ROWS_PER_W // 2)
            def _():
                pltpu.make_async_copy(
                    p_hbm.at[idx_v.at[j0 + 2]], rows0, sem0).start()

            pltpu.make_async_copy(p_hbm.at[idx_v.at[j1]], rows1, sem1).wait()
            pltpu.sync_copy(rows1, out_hbm.at[pl.ds((base + j1) * GROW, GROW)])
            return carry

        lax.fori_loop(0, ROWS_PER_W // 2, body, 0)

    return k(P, idx2d)


def kernel(input, table, W):
    idx2d = input.reshape(IDX_ROWS, GROW).astype(jnp.int32)
    P = _tc_project_table(table.T, W.T)
    out = _sc_gather(P, idx2d)
    return out.reshape(BATCH, HIST, SIZE)
